# baseline (speedup 1.0000x reference)
import jax
import jax.numpy as jnp
from jax import lax
from jax.experimental import pallas as pl
from jax.experimental.pallas import tpu as pltpu
from jax.experimental.pallas import tpu_sc as plsc

VOCAB = 1000000
DIM = 32
B = 4096
L = 200

_INFO = plsc.get_sparse_core_info()
_NC, _NS = _INFO.num_cores, _INFO.num_subcores
_NW = _NC * _NS
_BATCH = 128
_TOTAL = B * L
_NGROUP = _TOTAL // _BATCH
_NCHUNK = 8
_GPC = _NGROUP // _NCHUNK
_GPW = _GPC // _NW
_BPC = B // _NCHUNK


def _gather_body(ids_hbm, table_hbm, out_hbm, idx_v, rows_v, sem):
    wid = lax.axis_index("s") * _NC + lax.axis_index("c")
    base_g = wid * _GPW
    pltpu.sync_copy(ids_hbm.at[pl.ds(base_g, _GPW)], idx_v)

    def step(j, carry):
        pltpu.async_copy(table_hbm.at[idx_v.at[j]], rows_v, sem).wait()
        pltpu.sync_copy(rows_v, out_hbm.at[base_g + j])
        return carry

    lax.fori_loop(0, _GPW, step, 0)


def _sc_gather_chunk(ids_chunk, table):
    mesh = plsc.VectorSubcoreMesh(core_axis_name="c", subcore_axis_name="s")
    return pl.kernel(
        _gather_body,
        out_type=jax.ShapeDtypeStruct((_GPC, _BATCH, 2 * DIM), jnp.float32),
        mesh=mesh,
        scratch_types=[
            pltpu.VMEM((_GPW, _BATCH), jnp.int32),
            pltpu.VMEM((_BATCH, 2 * DIM), jnp.float32),
            pltpu.SemaphoreType.DMA,
        ],
        compiler_params=pltpu.CompilerParams(use_tc_tiling_on_sc=False),
    )(ids_chunk, table)


def kernel(ids, table):
    ids_grouped = ids.reshape(_NGROUP, _BATCH)
    outs = []
    for i in range(_NCHUNK):
        raw = _sc_gather_chunk(ids_grouped[i * _GPC : (i + 1) * _GPC], table)
        emb = raw.reshape(_BPC, L, 2 * DIM)
        outs.append(lax.complex(emb[..., :DIM], emb[..., DIM:]))
    return jnp.concatenate(outs, axis=0)
